# baseline (speedup 1.0000x reference)
import functools

import jax
import jax.numpy as jnp
from jax import lax
from jax.experimental import pallas as pl
from jax.experimental.pallas import tpu as pltpu
from jax.experimental.pallas import tpu_sc as plsc

B1 = 32
B2 = 8
N = 8192
K_SC = 8
LANES = 16
CHAINS = 4
GROUP = CHAINS * LANES
STEPS = N // GROUP


def _dyn_gather(v, idx):
  return lax.gather(
      v, idx[:, None],
      lax.GatherDimensionNumbers(
          offset_dims=(), collapsed_slice_dims=(0,), start_index_map=(0,)),
      (1,), mode=lax.GatherScatterMode.PROMISE_IN_BOUNDS)


def _lex_merge(v, i, v2, i2):
  take = (v2 < v) | ((v2 == v) & (i2 < i))
  return jnp.where(take, v2, v), jnp.where(take, i2, i)


def _sc_kernel(x_hbm, out_hbm, buf, res_v, sem):
  cid = lax.axis_index("c")
  sid = lax.axis_index("s")
  wid = sid * 2 + cid
  iota = lax.iota(jnp.int32, LANES)

  @pl.when(wid < K_SC)
  def _():
    for r in range(B2):
      pltpu.make_async_copy(x_hbm.at[wid, r], buf.at[r], sem).start()

    def row_body(r, res):
      pltpu.make_async_copy(x_hbm.at[wid, r], buf.at[r], sem).wait()

      def body(i, carry):
        minvs, minis = carry
        step = jnp.full((LANES,), i, jnp.int32)
        new_v, new_i = [], []
        for j in range(CHAINS):
          v = buf[r, pl.ds(i * GROUP + j * LANES, LANES)]
          m = v < minvs[j]
          new_v.append(jnp.minimum(v, minvs[j]))
          new_i.append(jnp.where(m, step, minis[j]))
        return tuple(new_v), tuple(new_i)

      init = (tuple(jnp.full((LANES,), jnp.inf, jnp.float32)
                    for _ in range(CHAINS)),
              tuple(jnp.zeros((LANES,), jnp.int32) for _ in range(CHAINS)))
      minvs, minis = lax.fori_loop(0, STEPS, body, init, unroll=4)

      mvs = list(minvs)
      mis = [minis[j] * GROUP + (j * LANES) + iota for j in range(CHAINS)]
      width = CHAINS
      while width > 1:
        half = width // 2
        for j in range(half):
          mvs[j], mis[j] = _lex_merge(mvs[j], mis[j],
                                      mvs[j + half], mis[j + half])
        width = half
      mv, mi = mvs[0], mis[0]

      for d in (8, 4, 2, 1):
        perm = (iota + d) & (LANES - 1)
        mv, mi = _lex_merge(mv, mi, _dyn_gather(mv, perm),
                            _dyn_gather(mi, perm))

      return jnp.where(iota == r, mi, res)

    res = lax.fori_loop(0, B2, row_body, jnp.zeros((LANES,), jnp.int32))
    res_v[...] = res
    pltpu.sync_copy(res_v.at[pl.ds(0, B2)],
                    out_hbm.at[pl.ds(wid * B2, B2)])


def _tc_kernel(x_ref, o_ref):
  data = x_ref[0]
  mn = jnp.min(data, axis=1, keepdims=True)
  ii = lax.broadcasted_iota(jnp.int32, (B2, N), 1)
  cand = jnp.where(data == mn, ii, N)
  o_ref[0, 0, :] = jnp.min(cand, axis=1)


@jax.jit
def kernel(x):
  mesh = plsc.VectorSubcoreMesh(core_axis_name="c", subcore_axis_name="s")
  sc_run = functools.partial(
      pl.kernel,
      mesh=mesh,
      out_type=jax.ShapeDtypeStruct((K_SC * B2,), jnp.int32),
      scratch_types=[
          pltpu.VMEM((B2, N), jnp.float32),
          pltpu.VMEM((LANES,), jnp.int32),
          pltpu.SemaphoreType.DMA,
      ],
  )(_sc_kernel)
  sc_out = sc_run(x)

  tc_out = pl.pallas_call(
      _tc_kernel,
      grid=(B1 - K_SC,),
      in_specs=[pl.BlockSpec((1, B2, N), lambda i: (i + K_SC, 0, 0))],
      out_specs=pl.BlockSpec((1, 1, B2), lambda i: (i, 0, 0)),
      out_shape=jax.ShapeDtypeStruct((B1 - K_SC, 1, B2), jnp.int32),
  )(x)

  return jnp.concatenate(
      [sc_out.reshape(K_SC, B2), tc_out.reshape(B1 - K_SC, B2)], axis=0)


# speedup vs baseline: 1.3157x; 1.3157x over previous
import functools

import jax
import jax.numpy as jnp
from jax import lax
from jax.experimental import pallas as pl
from jax.experimental.pallas import tpu as pltpu
from jax.experimental.pallas import tpu_sc as plsc

B1 = 32
B2 = 8
N = 8192
K_SC = 24
LANES = 16
NSUB = 16
NW = 32
RPW = K_SC * B2 // NW
CHAINS = 4
GROUP = CHAINS * LANES
STEPS = N // GROUP


def _dyn_gather(v, idx):
  return lax.gather(
      v, idx[:, None],
      lax.GatherDimensionNumbers(
          offset_dims=(), collapsed_slice_dims=(0,), start_index_map=(0,)),
      (1,), mode=lax.GatherScatterMode.PROMISE_IN_BOUNDS)


def _lex_merge(v, i, v2, i2):
  take = (v2 < v) | ((v2 == v) & (i2 < i))
  return jnp.where(take, v2, v), jnp.where(take, i2, i)


def _sc_kernel(x_hbm, out_hbm, buf, res_v, shared, sem):
  cid = lax.axis_index("c")
  sid = lax.axis_index("s")
  wid = cid * NSUB + sid
  f0 = wid * RPW
  iota = lax.iota(jnp.int32, LANES)

  for q in range(RPW):
    f = f0 + q
    pltpu.make_async_copy(x_hbm.at[f // B2, f % B2], buf.at[q], sem).start()

  res = jnp.zeros((LANES,), jnp.int32)
  for q in range(RPW):
    f = f0 + q
    pltpu.make_async_copy(x_hbm.at[f // B2, f % B2], buf.at[q], sem).wait()

    def body(i, carry):
      minvs, minis = carry
      step = jnp.full((LANES,), i, jnp.int32)
      new_v, new_i = [], []
      for j in range(CHAINS):
        v = buf[q, pl.ds(i * GROUP + j * LANES, LANES)]
        m = v < minvs[j]
        new_v.append(jnp.minimum(v, minvs[j]))
        new_i.append(jnp.where(m, step, minis[j]))
      return tuple(new_v), tuple(new_i)

    init = (tuple(jnp.full((LANES,), jnp.inf, jnp.float32)
                  for _ in range(CHAINS)),
            tuple(jnp.zeros((LANES,), jnp.int32) for _ in range(CHAINS)))
    minvs, minis = lax.fori_loop(0, STEPS, body, init, unroll=4)

    mvs = list(minvs)
    mis = [minis[j] * GROUP + (j * LANES) + iota for j in range(CHAINS)]
    width = CHAINS
    while width > 1:
      half = width // 2
      for j in range(half):
        mvs[j], mis[j] = _lex_merge(mvs[j], mis[j],
                                    mvs[j + half], mis[j + half])
      width = half
    mv, mi = mvs[0], mis[0]

    for d in (8, 4, 2, 1):
      perm = (iota + d) & (LANES - 1)
      mv, mi = _lex_merge(mv, mi, _dyn_gather(mv, perm),
                          _dyn_gather(mi, perm))

    res = jnp.where(iota == q, mi, res)

  res_v[...] = res
  pltpu.sync_copy(res_v.at[pl.ds(0, RPW)], shared.at[sid])
  plsc.subcore_barrier()

  @pl.when(sid == 0)
  def _():
    pltpu.sync_copy(shared, out_hbm.at[pl.ds(cid * NSUB, NSUB)])


def _tc_kernel(x_ref, o_ref):
  data = x_ref[0]
  mn = jnp.min(data, axis=1, keepdims=True)
  ii = lax.broadcasted_iota(jnp.int32, (B2, N), 1)
  cand = jnp.where(data == mn, ii, N)
  o_ref[0, 0, :] = jnp.min(cand, axis=1)


@jax.jit
def kernel(x):
  mesh = plsc.VectorSubcoreMesh(core_axis_name="c", subcore_axis_name="s")
  sc_run = functools.partial(
      pl.kernel,
      mesh=mesh,
      out_type=jax.ShapeDtypeStruct((NW, RPW), jnp.int32),
      scratch_types=[
          pltpu.VMEM((RPW, N), jnp.float32),
          pltpu.VMEM((LANES,), jnp.int32),
          pltpu.VMEM_SHARED((NSUB, RPW), jnp.int32),
          pltpu.SemaphoreType.DMA,
      ],
  )(_sc_kernel)
  sc_out = sc_run(x)

  tc_out = pl.pallas_call(
      _tc_kernel,
      grid=(B1 - K_SC,),
      in_specs=[pl.BlockSpec((1, B2, N), lambda i: (i + K_SC, 0, 0))],
      out_specs=pl.BlockSpec((1, 1, B2), lambda i: (i, 0, 0)),
      out_shape=jax.ShapeDtypeStruct((B1 - K_SC, 1, B2), jnp.int32),
  )(x)

  return jnp.concatenate(
      [sc_out.reshape(K_SC, B2), tc_out.reshape(B1 - K_SC, B2)], axis=0)


# speedup vs baseline: 1.3239x; 1.0063x over previous
import functools

import jax
import jax.numpy as jnp
from jax import lax
from jax.experimental import pallas as pl
from jax.experimental.pallas import tpu as pltpu
from jax.experimental.pallas import tpu_sc as plsc

B1 = 32
B2 = 8
N = 8192
K_SC = 24
LANES = 16
NSUB = 16
NW = 32
RPW = K_SC * B2 // NW
CHAINS = 4
GROUP = CHAINS * LANES
STEPS = N // GROUP


def _dyn_gather(v, idx):
  return lax.gather(
      v, idx[:, None],
      lax.GatherDimensionNumbers(
          offset_dims=(), collapsed_slice_dims=(0,), start_index_map=(0,)),
      (1,), mode=lax.GatherScatterMode.PROMISE_IN_BOUNDS)


def _lex_merge(v, i, v2, i2):
  take = (v2 < v) | ((v2 == v) & (i2 < i))
  return jnp.where(take, v2, v), jnp.where(take, i2, i)


def _sc_kernel(x_hbm, out_hbm, buf, res_v, shared, sem):
  cid = lax.axis_index("c")
  sid = lax.axis_index("s")
  wid = cid * NSUB + sid
  f0 = wid * RPW
  iota = lax.iota(jnp.int32, LANES)

  for q in range(RPW):
    f = f0 + q
    pltpu.make_async_copy(x_hbm.at[f // B2, f % B2], buf.at[q], sem).start()

  res = jnp.zeros((LANES,), jnp.int32)
  for q in range(RPW):
    f = f0 + q
    pltpu.make_async_copy(x_hbm.at[f // B2, f % B2], buf.at[q], sem).wait()

    def body(i, carry):
      minvs, minis = carry
      step = jnp.full((LANES,), i, jnp.int32)
      new_v, new_i = [], []
      for j in range(CHAINS):
        v = buf[q, pl.ds(i * GROUP + j * LANES, LANES)]
        m = v < minvs[j]
        new_v.append(jnp.minimum(v, minvs[j]))
        new_i.append(jnp.where(m, step, minis[j]))
      return tuple(new_v), tuple(new_i)

    init = (tuple(jnp.full((LANES,), jnp.inf, jnp.float32)
                  for _ in range(CHAINS)),
            tuple(jnp.zeros((LANES,), jnp.int32) for _ in range(CHAINS)))
    minvs, minis = lax.fori_loop(0, STEPS, body, init, unroll=4)

    mvs = list(minvs)
    mis = [minis[j] * GROUP + (j * LANES) + iota for j in range(CHAINS)]
    width = CHAINS
    while width > 1:
      half = width // 2
      for j in range(half):
        mvs[j], mis[j] = _lex_merge(mvs[j], mis[j],
                                    mvs[j + half], mis[j + half])
      width = half
    mv, mi = mvs[0], mis[0]

    for d in (8, 4, 2, 1):
      perm = (iota + d) & (LANES - 1)
      mv, mi = _lex_merge(mv, mi, _dyn_gather(mv, perm),
                          _dyn_gather(mi, perm))

    res = jnp.where(iota == q, mi, res)

  res_v[...] = res
  pltpu.sync_copy(res_v.at[pl.ds(0, RPW)], shared.at[sid])
  plsc.subcore_barrier()

  @pl.when(sid == 0)
  def _():
    pltpu.sync_copy(shared, out_hbm.at[pl.ds(cid * NSUB, NSUB)])


def _tc_kernel(x_ref, o_ref):
  for b in range(2):
    data = x_ref[b]
    mn = jnp.min(data, axis=1, keepdims=True)
    ii = lax.broadcasted_iota(jnp.int32, (B2, N), 1)
    cand = jnp.where(data == mn, ii, N)
    o_ref[b, 0, :] = jnp.min(cand, axis=1)


@jax.jit
def kernel(x):
  mesh = plsc.VectorSubcoreMesh(core_axis_name="c", subcore_axis_name="s")
  sc_run = functools.partial(
      pl.kernel,
      mesh=mesh,
      out_type=jax.ShapeDtypeStruct((NW, RPW), jnp.int32),
      scratch_types=[
          pltpu.VMEM((RPW, N), jnp.float32),
          pltpu.VMEM((LANES,), jnp.int32),
          pltpu.VMEM_SHARED((NSUB, RPW), jnp.int32),
          pltpu.SemaphoreType.DMA,
      ],
  )(_sc_kernel)
  sc_out = sc_run(x)

  tc_out = pl.pallas_call(
      _tc_kernel,
      grid=((B1 - K_SC) // 2,),
      in_specs=[pl.BlockSpec((2, B2, N), lambda i: (i + K_SC // 2, 0, 0))],
      out_specs=pl.BlockSpec((2, 1, B2), lambda i: (i, 0, 0)),
      out_shape=jax.ShapeDtypeStruct((B1 - K_SC, 1, B2), jnp.int32),
  )(x)

  return jnp.concatenate(
      [sc_out.reshape(K_SC, B2), tc_out.reshape(B1 - K_SC, B2)], axis=0)


# speedup vs baseline: 1.3650x; 1.0311x over previous
import functools

import jax
import jax.numpy as jnp
from jax import lax
from jax.experimental import pallas as pl
from jax.experimental.pallas import tpu as pltpu
from jax.experimental.pallas import tpu_sc as plsc

B1 = 32
B2 = 8
N = 8192
K_SC = 16
LANES = 16
NSUB = 16
NW = 32
RPW = K_SC * B2 // NW
CHAINS = 4
GROUP = CHAINS * LANES
STEPS = N // GROUP


def _dyn_gather(v, idx):
  return lax.gather(
      v, idx[:, None],
      lax.GatherDimensionNumbers(
          offset_dims=(), collapsed_slice_dims=(0,), start_index_map=(0,)),
      (1,), mode=lax.GatherScatterMode.PROMISE_IN_BOUNDS)


def _lex_merge(v, i, v2, i2):
  take = (v2 < v) | ((v2 == v) & (i2 < i))
  return jnp.where(take, v2, v), jnp.where(take, i2, i)


def _sc_kernel(x_hbm, out_hbm, buf, res_v, shared, sem):
  cid = lax.axis_index("c")
  sid = lax.axis_index("s")
  wid = cid * NSUB + sid
  f0 = wid * RPW
  iota = lax.iota(jnp.int32, LANES)

  for q in range(RPW):
    f = f0 + q
    pltpu.make_async_copy(x_hbm.at[f // B2, f % B2], buf.at[q], sem).start()

  res = jnp.zeros((LANES,), jnp.int32)
  for q in range(RPW):
    f = f0 + q
    pltpu.make_async_copy(x_hbm.at[f // B2, f % B2], buf.at[q], sem).wait()

    def body(i, carry):
      minvs, minis = carry
      step = jnp.full((LANES,), i, jnp.int32)
      new_v, new_i = [], []
      for j in range(CHAINS):
        v = buf[q, pl.ds(i * GROUP + j * LANES, LANES)]
        m = v < minvs[j]
        new_v.append(jnp.minimum(v, minvs[j]))
        new_i.append(jnp.where(m, step, minis[j]))
      return tuple(new_v), tuple(new_i)

    init = (tuple(jnp.full((LANES,), jnp.inf, jnp.float32)
                  for _ in range(CHAINS)),
            tuple(jnp.zeros((LANES,), jnp.int32) for _ in range(CHAINS)))
    minvs, minis = lax.fori_loop(0, STEPS, body, init, unroll=4)

    mvs = list(minvs)
    mis = [minis[j] * GROUP + (j * LANES) + iota for j in range(CHAINS)]
    width = CHAINS
    while width > 1:
      half = width // 2
      for j in range(half):
        mvs[j], mis[j] = _lex_merge(mvs[j], mis[j],
                                    mvs[j + half], mis[j + half])
      width = half
    mv, mi = mvs[0], mis[0]

    for d in (8, 4, 2, 1):
      perm = (iota + d) & (LANES - 1)
      mv, mi = _lex_merge(mv, mi, _dyn_gather(mv, perm),
                          _dyn_gather(mi, perm))

    res = jnp.where(iota == q, mi, res)

  res_v[...] = res
  pltpu.sync_copy(res_v.at[pl.ds(0, RPW)], shared.at[sid])
  plsc.subcore_barrier()

  @pl.when(sid == 0)
  def _():
    pltpu.sync_copy(shared, out_hbm.at[pl.ds(cid * NSUB, NSUB)])


def _tc_kernel(x_ref, o_ref):
  for b in range(2):
    data = x_ref[b]
    mn = jnp.min(data, axis=1, keepdims=True)
    ii = lax.broadcasted_iota(jnp.int32, (B2, N), 1)
    cand = jnp.where(data == mn, ii, N)
    o_ref[b, 0, :] = jnp.min(cand, axis=1)


@jax.jit
def kernel(x):
  mesh = plsc.VectorSubcoreMesh(core_axis_name="c", subcore_axis_name="s")
  sc_run = functools.partial(
      pl.kernel,
      mesh=mesh,
      out_type=jax.ShapeDtypeStruct((NW, RPW), jnp.int32),
      scratch_types=[
          pltpu.VMEM((RPW, N), jnp.float32),
          pltpu.VMEM((LANES,), jnp.int32),
          pltpu.VMEM_SHARED((NSUB, RPW), jnp.int32),
          pltpu.SemaphoreType.DMA,
      ],
  )(_sc_kernel)
  sc_out = sc_run(x)

  tc_out = pl.pallas_call(
      _tc_kernel,
      grid=((B1 - K_SC) // 2,),
      in_specs=[pl.BlockSpec((2, B2, N), lambda i: (i + K_SC // 2, 0, 0))],
      out_specs=pl.BlockSpec((2, 1, B2), lambda i: (i, 0, 0)),
      out_shape=jax.ShapeDtypeStruct((B1 - K_SC, 1, B2), jnp.int32),
  )(x)

  return jnp.concatenate(
      [sc_out.reshape(K_SC, B2), tc_out.reshape(B1 - K_SC, B2)], axis=0)
